# baseline (speedup 1.0000x reference)
import functools

import jax
import jax.numpy as jnp
from jax import lax
from jax.experimental import pallas as pl
from jax.experimental.pallas import tpu as pltpu
from jax.experimental.pallas import tpu_sc as plsc

_N = 10000
_D = 128
_E = 320000
_G = 64

_NC = 2
_NS = 16
_NW = _NC * _NS
_CHUNK = 128
_CPT = 80
_EPT = _CPT * _CHUNK
_XROWS = 10008
_AGG_ROWS = 10240
_ZROWS = _AGG_ROWS // _NS

_BLK = 2000
_NBLK = _N // _BLK


def _make_agg_kernel():
    mesh = plsc.VectorSubcoreMesh(core_axis_name="c", subcore_axis_name="s")

    @functools.partial(
        pl.kernel,
        mesh=mesh,
        out_type=jax.ShapeDtypeStruct((_NC, _AGG_ROWS, _D), jnp.float32),
        scratch_types=[
            pltpu.VMEM((_EPT,), jnp.int32),
            pltpu.VMEM((2, _CHUNK), jnp.int32),
            pltpu.VMEM((2, _CHUNK), jnp.int32),
            pltpu.VMEM((_CHUNK, _D), jnp.float32),
            pltpu.VMEM((_CHUNK, _D), jnp.float32),
            pltpu.VMEM_SHARED((_AGG_ROWS, _D), jnp.float32),
            pltpu.SemaphoreType.DMA,
            pltpu.SemaphoreType.DMA,
            pltpu.SemaphoreType.DMA,
            pltpu.SemaphoreType.DMA,
        ],
    )
    def agg(x_hbm, packed_hbm, zeros_hbm, out_hbm,
            packed_v, src_v, dst_v, rows0, rows1, acc_sh,
            gsem0, gsem1, ssem0, ssem1):
        rows = (rows0, rows1)
        gsem = (gsem0, gsem1)
        ssem = (ssem0, ssem1)
        cid = lax.axis_index("c")
        sid = lax.axis_index("s")
        wid = cid * _NS + sid

        def unpack(c, b):
            for j in range(_CHUNK // 16):
                v = packed_v[pl.ds(c * _CHUNK + j * 16, 16)]
                src_v[b, pl.ds(j * 16, 16)] = jnp.bitwise_and(v, 0xFFFF)
                dst_v[b, pl.ds(j * 16, 16)] = lax.shift_right_logical(v, 16)

        def gather(c_idx_slot, b):
            pltpu.async_copy(x_hbm.at[src_v.at[b]], rows[b], gsem[b])

        pltpu.sync_copy(zeros_hbm, acc_sh.at[pl.ds(sid * _ZROWS, _ZROWS)])
        pltpu.sync_copy(packed_hbm.at[wid], packed_v)
        unpack(0, 0)
        gather(0, 0)
        plsc.subcore_barrier()

        def body(i, carry):
            for b in range(2):
                c = i * 2 + b
                o = 1 - b
                pltpu.make_async_copy(x_hbm.at[src_v.at[b]], rows[b],
                                      gsem[b]).wait()
                pltpu.async_copy(rows[b], acc_sh.at[dst_v.at[b]], ssem[b],
                                 add=True, priority=1)

                @pl.when(c > 0)
                def _():
                    pltpu.make_async_copy(rows[o], acc_sh.at[dst_v.at[o]],
                                          ssem[o]).wait()

                @pl.when(c + 1 < _CPT)
                def _():
                    unpack(c + 1, o)
                    gather(c + 1, o)
            return carry

        lax.fori_loop(0, _CPT // 2, body, 0)
        pltpu.make_async_copy(rows[1], acc_sh.at[dst_v.at[1]], ssem[1]).wait()
        plsc.subcore_barrier()
        pltpu.sync_copy(acc_sh.at[pl.ds(sid * _ZROWS, _ZROWS)],
                        out_hbm.at[cid, pl.ds(sid * _ZROWS, _ZROWS)])

    return agg


def _layer_body(h_ref, a0_ref, a1_ref, wa_ref, ba_ref, wb_ref, bb_ref, o_ref):
    z = h_ref[...] + a0_ref[0] + a1_ref[0]
    t = jnp.maximum(
        jnp.dot(z, wa_ref[...], preferred_element_type=jnp.float32) + ba_ref[...],
        0.0)
    o_ref[...] = jnp.maximum(
        jnp.dot(t, wb_ref[...], preferred_element_type=jnp.float32) + bb_ref[...],
        0.0)


def _layer_call(h, agg, Wa, ba, Wb, bb):
    return pl.pallas_call(
        _layer_body,
        grid=(_NBLK,),
        in_specs=[
            pl.BlockSpec((_BLK, _D), lambda i: (i, 0)),
            pl.BlockSpec((1, _BLK, _D), lambda i: (0, i, 0)),
            pl.BlockSpec((1, _BLK, _D), lambda i: (1, i, 0)),
            pl.BlockSpec((_D, _D), lambda i: (0, 0)),
            pl.BlockSpec((1, _D), lambda i: (0, 0)),
            pl.BlockSpec((_D, _D), lambda i: (0, 0)),
            pl.BlockSpec((1, _D), lambda i: (0, 0)),
        ],
        out_specs=pl.BlockSpec((_BLK, _D), lambda i: (i, 0)),
        out_shape=jax.ShapeDtypeStruct((_N, _D), jnp.float32),
    )(h, agg, agg, Wa, ba, Wb, bb)


def _final_body(h_ref, a0_ref, a1_ref, wa_ref, ba_ref, wb_ref, bb_ref,
                wfc_ref, bfc_ref, batch_ref, o_ref, sums_ref, cnts_ref):
    i = pl.program_id(0)

    @pl.when(i == 0)
    def _():
        sums_ref[...] = jnp.zeros_like(sums_ref)
        cnts_ref[...] = jnp.zeros_like(cnts_ref)

    z = h_ref[...] + a0_ref[0] + a1_ref[0]
    t = jnp.maximum(
        jnp.dot(z, wa_ref[...], preferred_element_type=jnp.float32) + ba_ref[...],
        0.0)
    h3 = jnp.maximum(
        jnp.dot(t, wb_ref[...], preferred_element_type=jnp.float32) + bb_ref[...],
        0.0)
    onehot = (batch_ref[...] ==
              lax.broadcasted_iota(jnp.int32, (1, _G), 1)).astype(jnp.float32)
    sums_ref[...] += lax.dot_general(
        onehot, h3, (((0,), (0,)), ((), ())), preferred_element_type=jnp.float32)
    cnts_ref[...] += lax.dot_general(
        onehot, jnp.ones_like(h3), (((0,), (0,)), ((), ())),
        preferred_element_type=jnp.float32)

    @pl.when(i == _NBLK - 1)
    def _():
        pooled = sums_ref[...] / jnp.maximum(cnts_ref[...], 1.0)
        o_ref[...] = (jnp.dot(pooled, wfc_ref[...],
                              preferred_element_type=jnp.float32) + bfc_ref[...])


def _final_call(h, agg, Wa, ba, Wb, bb, Wfc, bfc, batch2):
    return pl.pallas_call(
        _final_body,
        grid=(_NBLK,),
        in_specs=[
            pl.BlockSpec((_BLK, _D), lambda i: (i, 0)),
            pl.BlockSpec((1, _BLK, _D), lambda i: (0, i, 0)),
            pl.BlockSpec((1, _BLK, _D), lambda i: (1, i, 0)),
            pl.BlockSpec((_D, _D), lambda i: (0, 0)),
            pl.BlockSpec((1, _D), lambda i: (0, 0)),
            pl.BlockSpec((_D, _D), lambda i: (0, 0)),
            pl.BlockSpec((1, _D), lambda i: (0, 0)),
            pl.BlockSpec((_D, _D), lambda i: (0, 0)),
            pl.BlockSpec((1, _D), lambda i: (0, 0)),
            pl.BlockSpec((_BLK, 1), lambda i: (i, 0)),
        ],
        out_specs=pl.BlockSpec((_G, _D), lambda i: (0, 0)),
        out_shape=jax.ShapeDtypeStruct((_G, _D), jnp.float32),
        scratch_shapes=[
            pltpu.VMEM((_G, _D), jnp.float32),
            pltpu.VMEM((_G, _D), jnp.float32),
        ],
    )(h, agg, agg, Wa, ba, Wb, bb, Wfc, bfc, batch2)


def kernel(x, edge_index, batch,
           W1a, b1a, W1b, b1b,
           W2a, b2a, W2b, b2b,
           W3a, b3a, W3b, b3b,
           Wfc, bfc):
    src = edge_index[0].astype(jnp.int32)
    dst = edge_index[1].astype(jnp.int32)
    ppt = _EPT - _E // _NW
    packed = (src + dst * 65536).reshape(_NW, _E // _NW)
    wids = jnp.arange(_NW, dtype=jnp.int32)[:, None]
    trash = _N + (wids * 7 + jnp.arange(ppt, dtype=jnp.int32)[None, :]) % (_AGG_ROWS - _N)
    packed = jnp.concatenate([packed, _N + trash * 65536], axis=1)
    zeros = jnp.zeros((_ZROWS, _D), jnp.float32)
    xpad = jnp.zeros((_XROWS - _N, _D), jnp.float32)
    batch2 = batch.astype(jnp.int32).reshape(_N, 1)
    b1a2, b1b2 = b1a.reshape(1, _D), b1b.reshape(1, _D)
    b2a2, b2b2 = b2a.reshape(1, _D), b2b.reshape(1, _D)
    b3a2, b3b2 = b3a.reshape(1, _D), b3b.reshape(1, _D)
    bfc2 = bfc.reshape(1, _D)

    agg_fn = _make_agg_kernel()

    a1 = agg_fn(jnp.concatenate([x, xpad]), packed, zeros)
    h1 = _layer_call(x, a1, W1a, b1a2, W1b, b1b2)
    a2 = agg_fn(jnp.concatenate([h1, xpad]), packed, zeros)
    h2 = _layer_call(h1, a2, W2a, b2a2, W2b, b2b2)
    a3 = agg_fn(jnp.concatenate([h2, xpad]), packed, zeros)
    out = _final_call(h2, a3, W3a, b3a2, W3b, b3b2, Wfc, bfc2, batch2)
    return out


# speedup vs baseline: 1.3525x; 1.3525x over previous
import functools

import jax
import jax.numpy as jnp
from jax import lax
from jax.experimental import pallas as pl
from jax.experimental.pallas import tpu as pltpu
from jax.experimental.pallas import tpu_sc as plsc

_N = 10000
_D = 128
_E = 320000
_G = 64

_NC = 2
_NS = 16
_NW = _NC * _NS
_CHUNK = 128
_CPT = 79
_EPT = _CPT * _CHUNK
_AGG_ROWS = 10240
_ZROWS = _AGG_ROWS // _NS

_BLK = 2000
_NBLK = _N // _BLK


def _make_agg_kernel():
    mesh = plsc.VectorSubcoreMesh(core_axis_name="c", subcore_axis_name="s")

    @functools.partial(
        pl.kernel,
        mesh=mesh,
        out_type=jax.ShapeDtypeStruct((_NC, _AGG_ROWS, _D), jnp.float32),
        scratch_types=[
            pltpu.VMEM((_CPT, _CHUNK), jnp.int32),
            pltpu.VMEM((_CPT, _CHUNK), jnp.int32),
            pltpu.VMEM((_CHUNK, _D), jnp.float32),
            pltpu.VMEM_SHARED((_AGG_ROWS, _D), jnp.float32),
            pltpu.SemaphoreType.DMA,
        ],
    )
    def agg(x_hbm, src_hbm, dst_hbm, zeros_hbm, out_hbm,
            src_v, dst_v, rows_v, acc_sh, sem):
        cid = lax.axis_index("c")
        sid = lax.axis_index("s")
        wid = cid * _NS + sid
        pltpu.sync_copy(zeros_hbm, acc_sh.at[pl.ds(sid * _ZROWS, _ZROWS)])
        pltpu.sync_copy(src_hbm.at[wid], src_v)
        pltpu.sync_copy(dst_hbm.at[wid], dst_v)
        plsc.subcore_barrier()

        def body(c, carry):
            pltpu.async_copy(x_hbm.at[src_v.at[c]], rows_v, sem).wait()
            pltpu.sync_copy(rows_v, acc_sh.at[dst_v.at[c]], add=True)
            return carry

        lax.fori_loop(0, _CPT, body, 0)
        plsc.subcore_barrier()
        pltpu.sync_copy(acc_sh.at[pl.ds(sid * _ZROWS, _ZROWS)],
                        out_hbm.at[cid, pl.ds(sid * _ZROWS, _ZROWS)])

    return agg


def _layer_body(h_ref, a0_ref, a1_ref, wa_ref, ba_ref, wb_ref, bb_ref, o_ref):
    z = h_ref[...] + a0_ref[0] + a1_ref[0]
    t = jnp.maximum(
        jnp.dot(z, wa_ref[...], preferred_element_type=jnp.float32) + ba_ref[...],
        0.0)
    o_ref[...] = jnp.maximum(
        jnp.dot(t, wb_ref[...], preferred_element_type=jnp.float32) + bb_ref[...],
        0.0)


def _layer_call(h, agg, Wa, ba, Wb, bb):
    return pl.pallas_call(
        _layer_body,
        grid=(_NBLK,),
        in_specs=[
            pl.BlockSpec((_BLK, _D), lambda i: (i, 0)),
            pl.BlockSpec((1, _BLK, _D), lambda i: (0, i, 0)),
            pl.BlockSpec((1, _BLK, _D), lambda i: (1, i, 0)),
            pl.BlockSpec((_D, _D), lambda i: (0, 0)),
            pl.BlockSpec((1, _D), lambda i: (0, 0)),
            pl.BlockSpec((_D, _D), lambda i: (0, 0)),
            pl.BlockSpec((1, _D), lambda i: (0, 0)),
        ],
        out_specs=pl.BlockSpec((_BLK, _D), lambda i: (i, 0)),
        out_shape=jax.ShapeDtypeStruct((_N, _D), jnp.float32),
    )(h, agg, agg, Wa, ba, Wb, bb)


def _final_body(h_ref, a0_ref, a1_ref, wa_ref, ba_ref, wb_ref, bb_ref,
                wfc_ref, bfc_ref, batch_ref, o_ref, sums_ref, cnts_ref):
    i = pl.program_id(0)

    @pl.when(i == 0)
    def _():
        sums_ref[...] = jnp.zeros_like(sums_ref)
        cnts_ref[...] = jnp.zeros_like(cnts_ref)

    z = h_ref[...] + a0_ref[0] + a1_ref[0]
    t = jnp.maximum(
        jnp.dot(z, wa_ref[...], preferred_element_type=jnp.float32) + ba_ref[...],
        0.0)
    h3 = jnp.maximum(
        jnp.dot(t, wb_ref[...], preferred_element_type=jnp.float32) + bb_ref[...],
        0.0)
    onehot = (batch_ref[...] ==
              lax.broadcasted_iota(jnp.int32, (1, _G), 1)).astype(jnp.float32)
    sums_ref[...] += lax.dot_general(
        onehot, h3, (((0,), (0,)), ((), ())), preferred_element_type=jnp.float32)
    cnts_ref[...] += lax.dot_general(
        onehot, jnp.ones_like(h3), (((0,), (0,)), ((), ())),
        preferred_element_type=jnp.float32)

    @pl.when(i == _NBLK - 1)
    def _():
        pooled = sums_ref[...] / jnp.maximum(cnts_ref[...], 1.0)
        o_ref[...] = (jnp.dot(pooled, wfc_ref[...],
                              preferred_element_type=jnp.float32) + bfc_ref[...])


def _final_call(h, agg, Wa, ba, Wb, bb, Wfc, bfc, batch2):
    return pl.pallas_call(
        _final_body,
        grid=(_NBLK,),
        in_specs=[
            pl.BlockSpec((_BLK, _D), lambda i: (i, 0)),
            pl.BlockSpec((1, _BLK, _D), lambda i: (0, i, 0)),
            pl.BlockSpec((1, _BLK, _D), lambda i: (1, i, 0)),
            pl.BlockSpec((_D, _D), lambda i: (0, 0)),
            pl.BlockSpec((1, _D), lambda i: (0, 0)),
            pl.BlockSpec((_D, _D), lambda i: (0, 0)),
            pl.BlockSpec((1, _D), lambda i: (0, 0)),
            pl.BlockSpec((_D, _D), lambda i: (0, 0)),
            pl.BlockSpec((1, _D), lambda i: (0, 0)),
            pl.BlockSpec((_BLK, 1), lambda i: (i, 0)),
        ],
        out_specs=pl.BlockSpec((_G, _D), lambda i: (0, 0)),
        out_shape=jax.ShapeDtypeStruct((_G, _D), jnp.float32),
        scratch_shapes=[
            pltpu.VMEM((_G, _D), jnp.float32),
            pltpu.VMEM((_G, _D), jnp.float32),
        ],
    )(h, agg, agg, Wa, ba, Wb, bb, Wfc, bfc, batch2)


def kernel(x, edge_index, batch,
           W1a, b1a, W1b, b1b,
           W2a, b2a, W2b, b2b,
           W3a, b3a, W3b, b3b,
           Wfc, bfc):
    src = edge_index[0].astype(jnp.int32)
    dst = edge_index[1].astype(jnp.int32)
    ppt = _EPT - _E // _NW
    src = jnp.concatenate(
        [src.reshape(_NW, _E // _NW),
         jnp.zeros((_NW, ppt), jnp.int32)], axis=1).reshape(_NW, _CPT, _CHUNK)
    wids = jnp.arange(_NW, dtype=jnp.int32)[:, None]
    trash = _N + (wids * 7 + jnp.arange(ppt, dtype=jnp.int32)[None, :]) % (_AGG_ROWS - _N)
    dst = jnp.concatenate(
        [dst.reshape(_NW, _E // _NW), trash], axis=1).reshape(_NW, _CPT, _CHUNK)
    zeros = jnp.zeros((_ZROWS, _D), jnp.float32)
    batch2 = batch.astype(jnp.int32).reshape(_N, 1)
    b1a2, b1b2 = b1a.reshape(1, _D), b1b.reshape(1, _D)
    b2a2, b2b2 = b2a.reshape(1, _D), b2b.reshape(1, _D)
    b3a2, b3b2 = b3a.reshape(1, _D), b3b.reshape(1, _D)
    bfc2 = bfc.reshape(1, _D)

    agg_fn = _make_agg_kernel()

    a1 = agg_fn(x, src, dst, zeros)
    h1 = _layer_call(x, a1, W1a, b1a2, W1b, b1b2)
    a2 = agg_fn(h1, src, dst, zeros)
    h2 = _layer_call(h1, a2, W2a, b2a2, W2b, b2b2)
    a3 = agg_fn(h2, src, dst, zeros)
    out = _final_call(h2, a3, W3a, b3a2, W3b, b3b2, Wfc, bfc2, batch2)
    return out
